# baseline (speedup 1.0000x reference)
import functools

import jax
import jax.numpy as jnp
from jax.experimental import pallas as pl

_TOPK = 64

_T_TILE_ENC = 128
_L_TILE_ENC = 1536
_T_TILE_DEC = 2048
_L_TILE_DEC = 1536


def _topk_mask(pre, ones_bf, k, nbits):
    int_min = jnp.int32(-2147483648)
    su = jax.lax.bitcast_convert_type(pre, jnp.int32)
    su = jnp.where(su < 0, su ^ jnp.int32(0x7FFFFFFF), su)
    kf = jnp.float32(k)

    def body(i, cur):
        cand = cur | jnp.left_shift(jnp.int32(1), 31 - i)
        thr = cand ^ int_min
        mask_bf = (su >= thr).astype(jnp.bfloat16)
        cnt = jax.lax.dot_general(
            mask_bf, ones_bf, (((1,), (0,)), ((), ())),
            preferred_element_type=jnp.float32,
        )[:, :1]
        return jnp.where(cnt >= kf, cand, cur)

    cur = jax.lax.fori_loop(
        0, nbits, body, jnp.zeros((pre.shape[0], 1), jnp.int32)
    )
    thr = cur ^ int_min
    return jnp.where(su >= thr, jnp.maximum(pre, 0.0), 0.0)


def _enc_body(
    x_ref, we_ref, be_ref, bd_ref, ones_ref, pre_ref, sp_ref, *, k, l_tile, n_l
):
    l = pl.program_id(1)
    xc = x_ref[...] - bd_ref[...]
    acc = jnp.dot(xc, we_ref[...], preferred_element_type=jnp.float32)
    pre_ref[:, pl.ds(l * l_tile, l_tile)] = acc + be_ref[...]

    @pl.when(l == n_l - 1)
    def _():
        sp_ref[...] = _topk_mask(pre_ref[...], ones_ref[...], k, 30)


def _dec_body(sp_ref, wd_ref, bd_ref, out_ref):
    l = pl.program_id(1)

    @pl.when(l == 0)
    def _():
        out_ref[...] = jnp.broadcast_to(bd_ref[...], out_ref.shape)

    out_ref[...] += jnp.dot(
        sp_ref[...], wd_ref[...], preferred_element_type=jnp.float32
    )


def kernel(x, W_enc, b_enc, W_dec, b_dec):
    T, D = x.shape
    L = W_enc.shape[1]

    t_tile = min(_T_TILE_ENC, T)
    l_tile = min(_L_TILE_ENC, L)
    n_t, n_l = T // t_tile, L // l_tile

    pre, sparse = pl.pallas_call(
        functools.partial(_enc_body, k=_TOPK, l_tile=l_tile, n_l=n_l),
        grid=(n_t, n_l),
        in_specs=[
            pl.BlockSpec((t_tile, D), lambda t, l: (t, 0)),
            pl.BlockSpec((D, l_tile), lambda t, l: (0, l)),
            pl.BlockSpec((1, l_tile), lambda t, l: (0, l)),
            pl.BlockSpec((1, D), lambda t, l: (0, 0)),
            pl.BlockSpec((L, 128), lambda t, l: (0, 0)),
        ],
        out_specs=[
            pl.BlockSpec((t_tile, L), lambda t, l: (t, 0)),
            pl.BlockSpec((t_tile, L), lambda t, l: (t, 0)),
        ],
        out_shape=[jax.ShapeDtypeStruct((T, L), jnp.float32)] * 2,
    )(
        x,
        W_enc,
        b_enc.reshape(1, L),
        b_dec.reshape(1, D),
        jnp.ones((L, 128), jnp.bfloat16),
    )

    td_tile = min(_T_TILE_DEC, T)
    ld_tile = min(_L_TILE_DEC, L)
    recon = pl.pallas_call(
        _dec_body,
        grid=(T // td_tile, L // ld_tile),
        in_specs=[
            pl.BlockSpec((td_tile, ld_tile), lambda t, l: (t, l)),
            pl.BlockSpec((ld_tile, D), lambda t, l: (l, 0)),
            pl.BlockSpec((1, D), lambda t, l: (0, 0)),
        ],
        out_specs=pl.BlockSpec((td_tile, D), lambda t, l: (t, 0)),
        out_shape=jax.ShapeDtypeStruct((T, D), jnp.float32),
    )(sparse, W_dec, b_dec.reshape(1, D))

    return (recon, sparse, pre)


# speedup vs baseline: 1.1956x; 1.1956x over previous
import functools

import jax
import jax.numpy as jnp
from jax.experimental import pallas as pl

_TOPK = 64

_T_TILE_ENC = 128
_L_TILE_ENC = 1536
_T_TILE_DEC = 2048
_L_TILE_DEC = 1536


def _topk_mask(pre, k, nbits):
    int_min = jnp.int32(-2147483648)
    su = jax.lax.bitcast_convert_type(pre, jnp.int32)
    su = jnp.where(su < 0, su ^ jnp.int32(0x7FFFFFFF), su)

    def body(i, cur):
        cand = cur | jnp.left_shift(jnp.int32(1), 31 - i)
        thr = cand ^ int_min
        cnt = jnp.sum((su >= thr).astype(jnp.int32), axis=1, keepdims=True)
        return jnp.where(cnt >= k, cand, cur)

    cur = jax.lax.fori_loop(
        0, nbits, body, jnp.zeros((pre.shape[0], 1), jnp.int32)
    )
    thr = cur ^ int_min
    return jnp.where(su >= thr, jnp.maximum(pre, 0.0), 0.0)


def _enc_body(x_ref, we_ref, be_ref, bd_ref, pre_ref, sp_ref, *, k, l_tile, n_l):
    l = pl.program_id(1)
    xc = x_ref[...] - bd_ref[...]
    acc = jnp.dot(xc, we_ref[...], preferred_element_type=jnp.float32)
    pre_ref[:, pl.ds(l * l_tile, l_tile)] = acc + be_ref[...]

    @pl.when(l == n_l - 1)
    def _():
        sp_ref[...] = _topk_mask(pre_ref[...], k, 30)


def _dec_body(sp_ref, wd_ref, bd_ref, out_ref):
    l = pl.program_id(1)

    @pl.when(l == 0)
    def _():
        out_ref[...] = jnp.broadcast_to(bd_ref[...], out_ref.shape)

    out_ref[...] += jnp.dot(
        sp_ref[...], wd_ref[...], preferred_element_type=jnp.float32
    )


def kernel(x, W_enc, b_enc, W_dec, b_dec):
    T, D = x.shape
    L = W_enc.shape[1]

    t_tile = min(_T_TILE_ENC, T)
    l_tile = min(_L_TILE_ENC, L)
    n_t, n_l = T // t_tile, L // l_tile

    pre, sparse = pl.pallas_call(
        functools.partial(_enc_body, k=_TOPK, l_tile=l_tile, n_l=n_l),
        grid=(n_t, n_l),
        in_specs=[
            pl.BlockSpec((t_tile, D), lambda t, l: (t, 0)),
            pl.BlockSpec((D, l_tile), lambda t, l: (0, l)),
            pl.BlockSpec((1, l_tile), lambda t, l: (0, l)),
            pl.BlockSpec((1, D), lambda t, l: (0, 0)),
        ],
        out_specs=[
            pl.BlockSpec((t_tile, L), lambda t, l: (t, 0)),
            pl.BlockSpec((t_tile, L), lambda t, l: (t, 0)),
        ],
        out_shape=[jax.ShapeDtypeStruct((T, L), jnp.float32)] * 2,
    )(x, W_enc, b_enc.reshape(1, L), b_dec.reshape(1, D))

    td_tile = min(_T_TILE_DEC, T)
    ld_tile = min(_L_TILE_DEC, L)
    recon = pl.pallas_call(
        _dec_body,
        grid=(T // td_tile, L // ld_tile),
        in_specs=[
            pl.BlockSpec((td_tile, ld_tile), lambda t, l: (t, l)),
            pl.BlockSpec((ld_tile, D), lambda t, l: (l, 0)),
            pl.BlockSpec((1, D), lambda t, l: (0, 0)),
        ],
        out_specs=pl.BlockSpec((td_tile, D), lambda t, l: (t, 0)),
        out_shape=jax.ShapeDtypeStruct((T, D), jnp.float32),
    )(sparse, W_dec, b_dec.reshape(1, D))

    return (recon, sparse, pre)
